# initial kernel scaffold (unmeasured)
import jax
import jax.numpy as jnp
from jax import lax
from jax.experimental import pallas as pl
from jax.experimental.pallas import tpu as pltpu

N_DEV = 8


def _silu(y):
    return y * jax.nn.sigmoid(y)


def kernel(x, w_mat):
    m, _ = x.shape
    _, n = w_mat.shape
    chunk = m // N_DEV

    xb = x.astype(jnp.bfloat16)
    wb = w_mat.astype(jnp.bfloat16)

    def body(x_ref, w_ref, out_ref, comm_ref, acc_ref, send_sems, recv_sems,
             copy_sem):
        my = lax.axis_index("i")
        left = (my + N_DEV - 1) % N_DEV
        right = (my + 1) % N_DEV

        barrier = pltpu.get_barrier_semaphore()
        for nbr in (left, right):
            pl.semaphore_signal(barrier, inc=1, device_id=(nbr,),
                                device_id_type=pl.DeviceIdType.MESH)
        pl.semaphore_wait(barrier, 2)

        def partial_into_acc(c):
            acc_ref[...] = jnp.dot(
                x_ref[pl.ds(c * chunk, chunk), :], w_ref[...],
                preferred_element_type=jnp.float32)

        partial_into_acc(left)
        comm_ref[0, ...] = acc_ref[...].astype(jnp.bfloat16)
        for s in range(N_DEV - 1):
            send_slot = s % 2
            recv_slot = (s + 1) % 2
            rdma = pltpu.make_async_remote_copy(
                src_ref=comm_ref.at[send_slot],
                dst_ref=comm_ref.at[recv_slot],
                send_sem=send_sems.at[send_slot],
                recv_sem=recv_sems.at[recv_slot],
                device_id=(right,),
                device_id_type=pl.DeviceIdType.MESH,
            )
            rdma.start()
            rdma.wait()
            c_recv = (my + 2 * N_DEV - 2 - s) % N_DEV
            partial_into_acc(c_recv)
            acc_ref[...] = acc_ref[...] + comm_ref[recv_slot].astype(jnp.float32)
            if s < N_DEV - 2:
                comm_ref[recv_slot, ...] = acc_ref[...].astype(jnp.bfloat16)
            else:
                comm_ref[recv_slot, ...] = _silu(acc_ref[...]).astype(jnp.bfloat16)

        own_slot = (N_DEV - 1) % 2
        cp = pltpu.make_async_copy(
            comm_ref.at[own_slot],
            out_ref.at[pl.ds(my * chunk, chunk), :],
            copy_sem)
        cp.start()
        cp.wait()

        for t in range(N_DEV - 1):
            send_slot = (own_slot + t) % 2
            recv_slot = (own_slot + t + 1) % 2
            rdma = pltpu.make_async_remote_copy(
                src_ref=comm_ref.at[send_slot],
                dst_ref=comm_ref.at[recv_slot],
                send_sem=send_sems.at[send_slot],
                recv_sem=recv_sems.at[recv_slot],
                device_id=(right,),
                device_id_type=pl.DeviceIdType.MESH,
            )
            rdma.start()
            rdma.wait()
            c_recv = (my + N_DEV - 1 - t) % N_DEV
            cp = pltpu.make_async_copy(
                comm_ref.at[recv_slot],
                out_ref.at[pl.ds(c_recv * chunk, chunk), :],
                copy_sem)
            cp.start()
            cp.wait()

    return pl.pallas_call(
        body,
        out_shape=jax.ShapeDtypeStruct((m, n), jnp.bfloat16),
        in_specs=[pl.BlockSpec(memory_space=pltpu.VMEM),
                  pl.BlockSpec(memory_space=pltpu.VMEM)],
        out_specs=pl.BlockSpec(memory_space=pltpu.ANY),
        scratch_shapes=[
            pltpu.VMEM((2, chunk, n), jnp.bfloat16),
            pltpu.VMEM((chunk, n), jnp.float32),
            pltpu.SemaphoreType.DMA((2,)),
            pltpu.SemaphoreType.DMA((2,)),
            pltpu.SemaphoreType.DMA,
        ],
        compiler_params=pltpu.CompilerParams(collective_id=0),
    )(xb, wb)


# baseline (device time: 1464939 ns/iter reference)
import jax
import jax.numpy as jnp
from jax import lax
from jax.experimental import pallas as pl
from jax.experimental.pallas import tpu as pltpu

N_DEV = 8


def _silu(y):
    return y * jax.nn.sigmoid(y)


def kernel(x, w_mat):
    m, _ = x.shape
    _, n = w_mat.shape
    chunk = m // N_DEV

    xb = x.astype(jnp.bfloat16)
    wb = w_mat.astype(jnp.bfloat16)

    def body(x_ref, w_ref, out_ref, comm_ref, send_sems, recv_sems,
             copy_sem):
        my = lax.axis_index("i")
        left = (my + N_DEV - 1) % N_DEV
        right = (my + 1) % N_DEV

        barrier = pltpu.get_barrier_semaphore()
        for nbr in (left, right):
            pl.semaphore_signal(barrier, inc=1, device_id=(nbr,),
                                device_id_type=pl.DeviceIdType.MESH)
        pl.semaphore_wait(barrier, 2)

        n_sub = 2
        sub = chunk // n_sub

        def accum_chunk(c, slot, first=False, act=False):
            for i in range(n_sub):
                part = jnp.dot(
                    x_ref[pl.ds(c * chunk + i * sub, sub), :], w_ref[...],
                    preferred_element_type=jnp.float32)
                if not first:
                    part = part + comm_ref[slot, pl.ds(i * sub, sub), :].astype(
                        jnp.float32)
                if act:
                    part = _silu(part)
                comm_ref[slot, pl.ds(i * sub, sub), :] = part.astype(jnp.bfloat16)

        def hop(send_slot, recv_slot):
            rdma = pltpu.make_async_remote_copy(
                src_ref=comm_ref.at[send_slot],
                dst_ref=comm_ref.at[recv_slot],
                send_sem=send_sems.at[send_slot],
                recv_sem=recv_sems.at[recv_slot],
                device_id=(right,),
                device_id_type=pl.DeviceIdType.MESH,
            )
            rdma.start()
            rdma.wait()

        accum_chunk(left, 0, first=True)

        def rs_step(s, carry):
            send_slot = lax.rem(s, 2)
            recv_slot = lax.rem(s + 1, 2)
            hop(send_slot, recv_slot)
            c_recv = lax.rem(my + 2 * N_DEV - 2 - s, N_DEV)
            accum_chunk(c_recv, recv_slot)
            return carry

        lax.fori_loop(0, N_DEV - 2, rs_step, 0)

        s_last = N_DEV - 2
        own_slot = (N_DEV - 1) % 2
        hop(s_last % 2, own_slot)
        accum_chunk(my, own_slot, act=True)

        cp = pltpu.make_async_copy(
            comm_ref.at[own_slot],
            out_ref.at[pl.ds(my * chunk, chunk), :],
            copy_sem)
        cp.start()
        cp.wait()

        def ag_step(t, carry):
            send_slot = lax.rem(own_slot + t, 2)
            recv_slot = lax.rem(own_slot + t + 1, 2)
            hop(send_slot, recv_slot)
            c_recv = lax.rem(my + N_DEV - 1 - t, N_DEV)
            cp = pltpu.make_async_copy(
                comm_ref.at[recv_slot],
                out_ref.at[pl.ds(c_recv * chunk, chunk), :],
                copy_sem)
            cp.start()
            cp.wait()
            return carry

        lax.fori_loop(0, N_DEV - 1, ag_step, 0)

    return pl.pallas_call(
        body,
        out_shape=jax.ShapeDtypeStruct((m, n), jnp.bfloat16),
        in_specs=[pl.BlockSpec(memory_space=pltpu.VMEM),
                  pl.BlockSpec(memory_space=pltpu.VMEM)],
        out_specs=pl.BlockSpec(memory_space=pl.ANY),
        scratch_shapes=[
            pltpu.VMEM((2, chunk, n), jnp.bfloat16),
            pltpu.SemaphoreType.DMA((2,)),
            pltpu.SemaphoreType.DMA((2,)),
            pltpu.SemaphoreType.DMA,
        ],
        compiler_params=pltpu.CompilerParams(
            collective_id=0, vmem_limit_bytes=38 * 1024 * 1024),
    )(xb, wb)


# device time: 821860 ns/iter; 1.7825x vs baseline; 1.7825x over previous
import jax
import jax.numpy as jnp
from jax import lax
from jax.experimental import pallas as pl
from jax.experimental.pallas import tpu as pltpu

N_DEV = 8
N_ROUND = 2


def _silu(y):
    return y * jax.nn.sigmoid(y)


def kernel(x, w_mat):
    m, _ = x.shape
    _, n = w_mat.shape
    rows_r = m // N_ROUND
    chunk = rows_r // N_DEV
    half = n // 2

    xb = x.astype(jnp.bfloat16)
    wb = w_mat.astype(jnp.bfloat16)
    wf = wb[:, :half]
    wbk = wb[:, half:]

    def body(x_ref, wf_ref, wb_ref, out_ref, comm_f, comm_b, p_f, p_b,
             send_f, recv_f, send_b, recv_b, copy_sem):
        my = lax.axis_index("i")
        left = (my + N_DEV - 1) % N_DEV
        right = (my + 1) % N_DEV

        barrier = pltpu.get_barrier_semaphore()
        for nbr in (left, right):
            pl.semaphore_signal(barrier, inc=1, device_id=(nbr,),
                                device_id_type=pl.DeviceIdType.MESH)
        pl.semaphore_wait(barrier, 2)

        def mk_hop(comm, send_sems, recv_sems, dst, s):
            send_slot = lax.rem(s, 2)
            recv_slot = lax.rem(s + 1, 2)
            return pltpu.make_async_remote_copy(
                src_ref=comm.at[send_slot],
                dst_ref=comm.at[recv_slot],
                send_sem=send_sems.at[send_slot],
                recv_sem=recv_sems.at[recv_slot],
                device_id=(dst,),
                device_id_type=pl.DeviceIdType.MESH)

        def start_hop(s):
            mk_hop(comm_f, send_f, recv_f, right, s).start()
            mk_hop(comm_b, send_b, recv_b, left, s).start()

        def wait_hop(s):
            mk_hop(comm_f, send_f, recv_f, right, s).wait()
            mk_hop(comm_b, send_b, recv_b, left, s).wait()

        def round_body(base):
            def precompute(cf, cb):
                p_f[...] = jnp.dot(
                    x_ref[pl.ds(base + cf * chunk, chunk), :], wf_ref[...],
                    preferred_element_type=jnp.float32)
                p_b[...] = jnp.dot(
                    x_ref[pl.ds(base + cb * chunk, chunk), :], wb_ref[...],
                    preferred_element_type=jnp.float32)

            def add_into(slot, act=False):
                pf = comm_f[slot].astype(jnp.float32) + p_f[...]
                pb = comm_b[slot].astype(jnp.float32) + p_b[...]
                if act:
                    pf = _silu(pf)
                    pb = _silu(pb)
                comm_f[slot, ...] = pf.astype(jnp.bfloat16)
                comm_b[slot, ...] = pb.astype(jnp.bfloat16)

            def copy_descs(cf, cb, slot):
                return (
                    pltpu.make_async_copy(
                        comm_f.at[slot],
                        out_ref.at[pl.ds(base + cf * chunk, chunk),
                                   pl.ds(0, half)],
                        copy_sem.at[0]),
                    pltpu.make_async_copy(
                        comm_b.at[slot],
                        out_ref.at[pl.ds(base + cb * chunk, chunk),
                                   pl.ds(half, half)],
                        copy_sem.at[1]),
                )

            def copy_out(cf, cb, slot):
                for d in copy_descs(cf, cb, slot):
                    d.start()
                for d in copy_descs(cf, cb, slot):
                    d.wait()

            precompute(left, (my + 1) % N_DEV)
            comm_f[0, ...] = p_f[...].astype(jnp.bfloat16)
            comm_b[0, ...] = p_b[...].astype(jnp.bfloat16)
            start_hop(0)

            def rs_step(s, carry):
                cf = lax.rem(my + 2 * N_DEV - 2 - s, N_DEV)
                cb = lax.rem(my + 2 + s, N_DEV)
                precompute(cf, cb)
                wait_hop(s)
                add_into(lax.rem(s + 1, 2))
                start_hop(s + 1)
                return carry

            lax.fori_loop(0, N_DEV - 2, rs_step, 0)

            precompute(my, my)
            wait_hop(N_DEV - 2)
            own_slot = (N_DEV - 1) % 2
            add_into(own_slot, act=True)
            copy_out(my, my, own_slot)

            start_hop(N_DEV - 1)

            def ag_step(t, carry):
                wait_hop(N_DEV - 1 + t)
                recv_slot = lax.rem(N_DEV + t, 2)
                cf = lax.rem(my + 2 * N_DEV - 1 - t, N_DEV)
                cb = lax.rem(my + 1 + t, N_DEV)
                copy_out(cf, cb, recv_slot)
                return carry

            def ag_step_and_send(t, carry):
                ag_step(t, carry)
                start_hop(N_DEV + t)
                return carry

            lax.fori_loop(0, N_DEV - 2, ag_step_and_send, 0)
            ag_step(N_DEV - 2, 0)

        for r in range(N_ROUND):
            round_body(r * rows_r)

    return pl.pallas_call(
        body,
        out_shape=jax.ShapeDtypeStruct((m, n), jnp.bfloat16),
        in_specs=[pl.BlockSpec(memory_space=pltpu.VMEM),
                  pl.BlockSpec(memory_space=pltpu.VMEM),
                  pl.BlockSpec(memory_space=pltpu.VMEM)],
        out_specs=pl.BlockSpec(memory_space=pl.ANY),
        scratch_shapes=[
            pltpu.VMEM((2, chunk, half), jnp.bfloat16),
            pltpu.VMEM((2, chunk, half), jnp.bfloat16),
            pltpu.VMEM((chunk, half), jnp.float32),
            pltpu.VMEM((chunk, half), jnp.float32),
            pltpu.SemaphoreType.DMA((2,)),
            pltpu.SemaphoreType.DMA((2,)),
            pltpu.SemaphoreType.DMA((2,)),
            pltpu.SemaphoreType.DMA((2,)),
            pltpu.SemaphoreType.DMA((2,)),
        ],
        compiler_params=pltpu.CompilerParams(
            collective_id=0, vmem_limit_bytes=40 * 1024 * 1024),
    )(xb, wf, wbk)


# device time: 793974 ns/iter; 1.8451x vs baseline; 1.0351x over previous
import jax
import jax.numpy as jnp
from jax import lax
from jax.experimental import pallas as pl
from jax.experimental.pallas import tpu as pltpu

N_DEV = 8
N_ROUND = 2


def _silu(y):
    return y * jax.nn.sigmoid(y)


def kernel(x, w_mat):
    m, _ = x.shape
    _, n = w_mat.shape
    rows_r = m // N_ROUND
    chunk = rows_r // N_DEV
    half = n // 2

    xb = x.astype(jnp.bfloat16)
    wb = w_mat.astype(jnp.bfloat16)
    wf = wb[:, :half]
    wbk = wb[:, half:]

    def body(x_ref, wf_ref, wb_ref, out_ref, comm_f, comm_b, p_f, p_b,
             send_f, recv_f, send_b, recv_b, copy_sem):
        my = lax.axis_index("i")
        left = (my + N_DEV - 1) % N_DEV
        right = (my + 1) % N_DEV

        barrier = pltpu.get_barrier_semaphore()
        for nbr in (left, right):
            pl.semaphore_signal(barrier, inc=1, device_id=(nbr,),
                                device_id_type=pl.DeviceIdType.MESH)
        pl.semaphore_wait(barrier, 2)

        def mk_hop(comm, send_sems, recv_sems, dst, s):
            send_slot = lax.rem(s, 2)
            recv_slot = lax.rem(s + 1, 2)
            return pltpu.make_async_remote_copy(
                src_ref=comm.at[send_slot],
                dst_ref=comm.at[recv_slot],
                send_sem=send_sems.at[send_slot],
                recv_sem=recv_sems.at[recv_slot],
                device_id=(dst,),
                device_id_type=pl.DeviceIdType.MESH)

        def start_hop(s):
            mk_hop(comm_f, send_f, recv_f, right, s).start()
            mk_hop(comm_b, send_b, recv_b, left, s).start()

        def wait_hop(s):
            mk_hop(comm_f, send_f, recv_f, right, s).wait()
            mk_hop(comm_b, send_b, recv_b, left, s).wait()

        def round_body(base):
            def precompute(cf, cb):
                p_f[...] = jnp.dot(
                    x_ref[pl.ds(base + cf * chunk, chunk), :], wf_ref[...],
                    preferred_element_type=jnp.float32)
                p_b[...] = jnp.dot(
                    x_ref[pl.ds(base + cb * chunk, chunk), :], wb_ref[...],
                    preferred_element_type=jnp.float32)

            def add_into(slot, act=False):
                pf = comm_f[slot].astype(jnp.float32) + p_f[...]
                pb = comm_b[slot].astype(jnp.float32) + p_b[...]
                if act:
                    pf = _silu(pf)
                    pb = _silu(pb)
                comm_f[slot, ...] = pf.astype(jnp.bfloat16)
                comm_b[slot, ...] = pb.astype(jnp.bfloat16)

            def copy_descs(cf, cb, slot):
                return (
                    pltpu.make_async_copy(
                        comm_f.at[slot],
                        out_ref.at[pl.ds(base + cf * chunk, chunk),
                                   pl.ds(0, half)],
                        copy_sem.at[slot, 0]),
                    pltpu.make_async_copy(
                        comm_b.at[slot],
                        out_ref.at[pl.ds(base + cb * chunk, chunk),
                                   pl.ds(half, half)],
                        copy_sem.at[slot, 1]),
                )

            def copy_start(cf, cb, slot):
                for d in copy_descs(cf, cb, slot):
                    d.start()

            def copy_wait(cf, cb, slot):
                for d in copy_descs(cf, cb, slot):
                    d.wait()

            precompute(left, (my + 1) % N_DEV)
            comm_f[0, ...] = p_f[...].astype(jnp.bfloat16)
            comm_b[0, ...] = p_b[...].astype(jnp.bfloat16)
            start_hop(0)

            def rs_step(s, carry):
                cf = lax.rem(my + 2 * N_DEV - 2 - s, N_DEV)
                cb = lax.rem(my + 2 + s, N_DEV)
                precompute(cf, cb)
                wait_hop(s)
                add_into(lax.rem(s + 1, 2))
                start_hop(s + 1)
                return carry

            lax.fori_loop(0, N_DEV - 2, rs_step, 0)

            precompute(my, my)
            wait_hop(N_DEV - 2)
            own_slot = (N_DEV - 1) % 2
            add_into(own_slot, act=True)
            copy_start(my, my, own_slot)

            start_hop(N_DEV - 1)

            def ag_step(t, pcf, pcb):
                wait_hop(N_DEV - 1 + t)
                recv_slot = lax.rem(N_DEV + t, 2)
                cf = lax.rem(my + 2 * N_DEV - 1 - t, N_DEV)
                cb = lax.rem(my + 1 + t, N_DEV)
                copy_wait(pcf, pcb, lax.rem(t + 1, 2))
                copy_start(cf, cb, recv_slot)
                return cf, cb

            def ag_step_and_send(t, carry):
                cf, cb = ag_step(t, *carry)
                start_hop(N_DEV + t)
                return cf, cb

            carry = lax.fori_loop(0, N_DEV - 2, ag_step_and_send,
                                  (my, my))
            cf, cb = ag_step(N_DEV - 2, *carry)
            copy_wait(cf, cb, lax.rem(N_DEV - 2, 2))

        for r in range(N_ROUND):
            round_body(r * rows_r)

    return pl.pallas_call(
        body,
        out_shape=jax.ShapeDtypeStruct((m, n), jnp.bfloat16),
        in_specs=[pl.BlockSpec(memory_space=pltpu.VMEM),
                  pl.BlockSpec(memory_space=pltpu.VMEM),
                  pl.BlockSpec(memory_space=pltpu.VMEM)],
        out_specs=pl.BlockSpec(memory_space=pl.ANY),
        scratch_shapes=[
            pltpu.VMEM((2, chunk, half), jnp.bfloat16),
            pltpu.VMEM((2, chunk, half), jnp.bfloat16),
            pltpu.VMEM((chunk, half), jnp.float32),
            pltpu.VMEM((chunk, half), jnp.float32),
            pltpu.SemaphoreType.DMA((2,)),
            pltpu.SemaphoreType.DMA((2,)),
            pltpu.SemaphoreType.DMA((2,)),
            pltpu.SemaphoreType.DMA((2,)),
            pltpu.SemaphoreType.DMA((2, 2)),
        ],
        compiler_params=pltpu.CompilerParams(
            collective_id=0, vmem_limit_bytes=40 * 1024 * 1024),
    )(xb, wf, wbk)


# device time: 758066 ns/iter; 1.9325x vs baseline; 1.0474x over previous
import jax
import jax.numpy as jnp
from jax import lax
from jax.experimental import pallas as pl
from jax.experimental.pallas import tpu as pltpu

N_DEV = 8
N_ROUND = 2


def _silu(y):
    return y * jax.nn.sigmoid(y)


def kernel(x, w_mat):
    m, _ = x.shape
    _, n = w_mat.shape
    rows_r = m // N_ROUND
    chunk = rows_r // N_DEV
    half = n // 2

    xb = x.astype(jnp.bfloat16)
    wb = w_mat.astype(jnp.bfloat16)
    wf = wb[:, :half]
    wbk = wb[:, half:]

    def body(x_ref, wf_ref, wb_ref, out_ref, comm_f, comm_b, p_f, p_b,
             send_f, recv_f, send_b, recv_b, copy_sem):
        my = lax.axis_index("i")
        left = (my + N_DEV - 1) % N_DEV
        right = (my + 1) % N_DEV

        barrier = pltpu.get_barrier_semaphore()
        for nbr in (left, right):
            pl.semaphore_signal(barrier, inc=1, device_id=(nbr,),
                                device_id_type=pl.DeviceIdType.MESH)
        pl.semaphore_wait(barrier, 2)

        subr = chunk // 2

        def mk_hop(comm, send_sems, recv_sems, dst, s):
            send_slot = lax.rem(s, 2)
            recv_slot = lax.rem(s + 1, 2)
            return pltpu.make_async_remote_copy(
                src_ref=comm.at[send_slot],
                dst_ref=comm.at[recv_slot],
                send_sem=send_sems.at[send_slot, 0],
                recv_sem=recv_sems.at[recv_slot, 0],
                device_id=(dst,),
                device_id_type=pl.DeviceIdType.MESH)

        def start_hop(s):
            mk_hop(comm_f, send_f, recv_f, right, s).start()
            mk_hop(comm_b, send_b, recv_b, left, s).start()

        def wait_hop(s):
            mk_hop(comm_f, send_f, recv_f, right, s).wait()
            mk_hop(comm_b, send_b, recv_b, left, s).wait()

        def mk_hop_sub(comm, send_sems, recv_sems, dst, s, k):
            send_slot = lax.rem(s, 2)
            recv_slot = lax.rem(s + 1, 2)
            rows = pl.ds(k * subr, subr)
            return pltpu.make_async_remote_copy(
                src_ref=comm.at[send_slot, rows, :],
                dst_ref=comm.at[recv_slot, rows, :],
                send_sem=send_sems.at[send_slot, k],
                recv_sem=recv_sems.at[recv_slot, k],
                device_id=(dst,),
                device_id_type=pl.DeviceIdType.MESH)

        def start_hop_sub(s, k):
            mk_hop_sub(comm_f, send_f, recv_f, right, s, k).start()
            mk_hop_sub(comm_b, send_b, recv_b, left, s, k).start()

        def wait_hop_sub(s, k):
            mk_hop_sub(comm_f, send_f, recv_f, right, s, k).wait()
            mk_hop_sub(comm_b, send_b, recv_b, left, s, k).wait()

        def round_body(base):
            def precompute(cf, cb):
                p_f[...] = jnp.dot(
                    x_ref[pl.ds(base + cf * chunk, chunk), :], wf_ref[...],
                    preferred_element_type=jnp.float32)
                p_b[...] = jnp.dot(
                    x_ref[pl.ds(base + cb * chunk, chunk), :], wb_ref[...],
                    preferred_element_type=jnp.float32)

            def add_sub(slot, k, act=False):
                rows = pl.ds(k * subr, subr)
                pf = comm_f[slot, rows, :].astype(jnp.float32) + p_f[rows, :]
                pb = comm_b[slot, rows, :].astype(jnp.float32) + p_b[rows, :]
                if act:
                    pf = _silu(pf)
                    pb = _silu(pb)
                comm_f[slot, rows, :] = pf.astype(jnp.bfloat16)
                comm_b[slot, rows, :] = pb.astype(jnp.bfloat16)

            def copy_descs(cf, cb, slot):
                return (
                    pltpu.make_async_copy(
                        comm_f.at[slot],
                        out_ref.at[pl.ds(base + cf * chunk, chunk),
                                   pl.ds(0, half)],
                        copy_sem.at[slot, 0]),
                    pltpu.make_async_copy(
                        comm_b.at[slot],
                        out_ref.at[pl.ds(base + cb * chunk, chunk),
                                   pl.ds(half, half)],
                        copy_sem.at[slot, 1]),
                )

            def copy_start(cf, cb, slot):
                for d in copy_descs(cf, cb, slot):
                    d.start()

            def copy_wait(cf, cb, slot):
                for d in copy_descs(cf, cb, slot):
                    d.wait()

            precompute(left, (my + 1) % N_DEV)
            comm_f[0, ...] = p_f[...].astype(jnp.bfloat16)
            comm_b[0, ...] = p_b[...].astype(jnp.bfloat16)
            for k in range(2):
                start_hop_sub(0, k)

            def rs_step(s, carry):
                cf = lax.rem(my + 2 * N_DEV - 2 - s, N_DEV)
                cb = lax.rem(my + 2 + s, N_DEV)
                precompute(cf, cb)
                recv_slot = lax.rem(s + 1, 2)
                for k in range(2):
                    wait_hop_sub(s, k)
                    add_sub(recv_slot, k)
                    start_hop_sub(s + 1, k)
                return carry

            lax.fori_loop(0, N_DEV - 2, rs_step, 0)

            precompute(my, my)
            own_slot = (N_DEV - 1) % 2
            for k in range(2):
                wait_hop_sub(N_DEV - 2, k)
                add_sub(own_slot, k, act=True)
            copy_start(my, my, own_slot)

            start_hop(N_DEV - 1)

            def ag_step(t, pcf, pcb):
                wait_hop(N_DEV - 1 + t)
                recv_slot = lax.rem(N_DEV + t, 2)
                cf = lax.rem(my + 2 * N_DEV - 1 - t, N_DEV)
                cb = lax.rem(my + 1 + t, N_DEV)
                copy_wait(pcf, pcb, lax.rem(t + 1, 2))
                copy_start(cf, cb, recv_slot)
                return cf, cb

            def ag_step_and_send(t, carry):
                cf, cb = ag_step(t, *carry)
                start_hop(N_DEV + t)
                return cf, cb

            carry = lax.fori_loop(0, N_DEV - 2, ag_step_and_send,
                                  (my, my))
            cf, cb = ag_step(N_DEV - 2, *carry)
            copy_wait(cf, cb, lax.rem(N_DEV - 2, 2))

        for r in range(N_ROUND):
            round_body(r * rows_r)

    return pl.pallas_call(
        body,
        out_shape=jax.ShapeDtypeStruct((m, n), jnp.bfloat16),
        in_specs=[pl.BlockSpec(memory_space=pltpu.VMEM),
                  pl.BlockSpec(memory_space=pltpu.VMEM),
                  pl.BlockSpec(memory_space=pltpu.VMEM)],
        out_specs=pl.BlockSpec(memory_space=pl.ANY),
        scratch_shapes=[
            pltpu.VMEM((2, chunk, half), jnp.bfloat16),
            pltpu.VMEM((2, chunk, half), jnp.bfloat16),
            pltpu.VMEM((chunk, half), jnp.float32),
            pltpu.VMEM((chunk, half), jnp.float32),
            pltpu.SemaphoreType.DMA((2, 2)),
            pltpu.SemaphoreType.DMA((2, 2)),
            pltpu.SemaphoreType.DMA((2, 2)),
            pltpu.SemaphoreType.DMA((2, 2)),
            pltpu.SemaphoreType.DMA((2, 2)),
        ],
        compiler_params=pltpu.CompilerParams(
            collective_id=0, vmem_limit_bytes=40 * 1024 * 1024),
    )(xb, wf, wbk)


# device time: 739915 ns/iter; 1.9799x vs baseline; 1.0245x over previous
import jax
import jax.numpy as jnp
from jax import lax
from jax.experimental import pallas as pl
from jax.experimental.pallas import tpu as pltpu

N_DEV = 8
N_SUB = 8


def _silu(y):
    return y * jax.nn.sigmoid(y)


def kernel(x, w_mat):
    m, _ = x.shape
    _, n = w_mat.shape
    chunk = m // N_DEV
    rdir = chunk // 2
    subr = rdir // N_SUB
    dot_r = rdir // 4

    xb = x.astype(jnp.bfloat16)
    wb = w_mat.astype(jnp.bfloat16)

    def body(x_ref, w_ref, out_ref, comm_f, comm_b, p_f, p_b,
             send_f, recv_f, send_b, recv_b, copy_sem):
        my = lax.axis_index("i")
        left = (my + N_DEV - 1) % N_DEV
        right = (my + 1) % N_DEV

        barrier = pltpu.get_barrier_semaphore()
        for nbr in (left, right):
            pl.semaphore_signal(barrier, inc=1, device_id=(nbr,),
                                device_id_type=pl.DeviceIdType.MESH)
        pl.semaphore_wait(barrier, 2)

        def mk_hop(comm, send_sems, recv_sems, dst, s):
            send_slot = lax.rem(s, 2)
            recv_slot = lax.rem(s + 1, 2)
            return pltpu.make_async_remote_copy(
                src_ref=comm.at[send_slot],
                dst_ref=comm.at[recv_slot],
                send_sem=send_sems.at[send_slot, 0],
                recv_sem=recv_sems.at[recv_slot, 0],
                device_id=(dst,),
                device_id_type=pl.DeviceIdType.MESH)

        def start_hop(s):
            mk_hop(comm_f, send_f, recv_f, right, s).start()
            mk_hop(comm_b, send_b, recv_b, left, s).start()

        def wait_hop(s):
            mk_hop(comm_f, send_f, recv_f, right, s).wait()
            mk_hop(comm_b, send_b, recv_b, left, s).wait()

        def mk_hop_sub(comm, send_sems, recv_sems, dst, s, k):
            send_slot = lax.rem(s, 2)
            recv_slot = lax.rem(s + 1, 2)
            rows = pl.ds(k * subr, subr)
            return pltpu.make_async_remote_copy(
                src_ref=comm.at[send_slot, rows, :],
                dst_ref=comm.at[recv_slot, rows, :],
                send_sem=send_sems.at[send_slot, k],
                recv_sem=recv_sems.at[recv_slot, k],
                device_id=(dst,),
                device_id_type=pl.DeviceIdType.MESH)

        def start_hop_sub(s, k):
            mk_hop_sub(comm_f, send_f, recv_f, right, s, k).start()
            mk_hop_sub(comm_b, send_b, recv_b, left, s, k).start()

        def wait_hop_sub(s, k):
            mk_hop_sub(comm_f, send_f, recv_f, right, s, k).wait()
            mk_hop_sub(comm_b, send_b, recv_b, left, s, k).wait()

        def precompute(cf, cb):
            def itf(k, c):
                rows = pl.ds(k * dot_r, dot_r)
                p_f[rows, :] = jnp.dot(
                    x_ref[pl.ds(cf * chunk + k * dot_r, dot_r), :],
                    w_ref[...],
                    preferred_element_type=jnp.float32).astype(jnp.bfloat16)
                return c

            def itb(k, c):
                rows = pl.ds(k * dot_r, dot_r)
                p_b[rows, :] = jnp.dot(
                    x_ref[pl.ds(cb * chunk + rdir + k * dot_r, dot_r), :],
                    w_ref[...],
                    preferred_element_type=jnp.float32).astype(jnp.bfloat16)
                return c

            lax.fori_loop(0, rdir // dot_r, itf, 0)
            lax.fori_loop(0, rdir // dot_r, itb, 0)

        def add_sub(slot, k, act=False):
            rows = pl.ds(k * subr, subr)
            pf = (comm_f[slot, rows, :].astype(jnp.float32)
                  + p_f[rows, :].astype(jnp.float32))
            pb = (comm_b[slot, rows, :].astype(jnp.float32)
                  + p_b[rows, :].astype(jnp.float32))
            if act:
                pf = _silu(pf)
                pb = _silu(pb)
            comm_f[slot, rows, :] = pf.astype(jnp.bfloat16)
            comm_b[slot, rows, :] = pb.astype(jnp.bfloat16)

        def rs_subs(s, act=False):
            def it(k, carry):
                wait_hop_sub(s, k)
                add_sub(lax.rem(s + 1, 2), k, act=act)
                if not act:
                    start_hop_sub(s + 1, k)
                return carry
            lax.fori_loop(0, N_SUB, it, 0)

        def copy_descs(cf, cb, slot):
            return (
                pltpu.make_async_copy(
                    comm_f.at[slot],
                    out_ref.at[pl.ds(cf * chunk, rdir), :],
                    copy_sem.at[slot, 0]),
                pltpu.make_async_copy(
                    comm_b.at[slot],
                    out_ref.at[pl.ds(cb * chunk + rdir, rdir), :],
                    copy_sem.at[slot, 1]),
            )

        def copy_start(cf, cb, slot):
            for d in copy_descs(cf, cb, slot):
                d.start()

        def copy_wait(cf, cb, slot):
            for d in copy_descs(cf, cb, slot):
                d.wait()

        precompute(left, (my + 1) % N_DEV)
        comm_f[0, ...] = p_f[...]
        comm_b[0, ...] = p_b[...]
        for k in range(N_SUB):
            start_hop_sub(0, k)

        def rs_step(s, carry):
            cf = lax.rem(my + 2 * N_DEV - 2 - s, N_DEV)
            cb = lax.rem(my + 2 + s, N_DEV)
            precompute(cf, cb)
            rs_subs(s)
            return carry

        lax.fori_loop(0, N_DEV - 2, rs_step, 0)

        precompute(my, my)
        own_slot = (N_DEV - 1) % 2
        rs_subs(N_DEV - 2, act=True)
        copy_start(my, my, own_slot)

        start_hop(N_DEV - 1)

        def ag_step(t, pcf, pcb):
            wait_hop(N_DEV - 1 + t)
            recv_slot = lax.rem(N_DEV + t, 2)
            cf = lax.rem(my + 2 * N_DEV - 1 - t, N_DEV)
            cb = lax.rem(my + 1 + t, N_DEV)
            copy_wait(pcf, pcb, lax.rem(t + 1, 2))
            copy_start(cf, cb, recv_slot)
            return cf, cb

        def ag_step_and_send(t, carry):
            cf, cb = ag_step(t, *carry)
            start_hop(N_DEV + t)
            return cf, cb

        carry = lax.fori_loop(0, N_DEV - 2, ag_step_and_send, (my, my))
        cf, cb = ag_step(N_DEV - 2, *carry)
        copy_wait(cf, cb, lax.rem(N_DEV - 2, 2))

    return pl.pallas_call(
        body,
        out_shape=jax.ShapeDtypeStruct((m, n), jnp.bfloat16),
        in_specs=[pl.BlockSpec(memory_space=pltpu.VMEM),
                  pl.BlockSpec(memory_space=pltpu.VMEM)],
        out_specs=pl.BlockSpec(memory_space=pl.ANY),
        scratch_shapes=[
            pltpu.VMEM((2, rdir, n), jnp.bfloat16),
            pltpu.VMEM((2, rdir, n), jnp.bfloat16),
            pltpu.VMEM((rdir, n), jnp.bfloat16),
            pltpu.VMEM((rdir, n), jnp.bfloat16),
            pltpu.SemaphoreType.DMA((2, N_SUB)),
            pltpu.SemaphoreType.DMA((2, N_SUB)),
            pltpu.SemaphoreType.DMA((2, N_SUB)),
            pltpu.SemaphoreType.DMA((2, N_SUB)),
            pltpu.SemaphoreType.DMA((2, 2)),
        ],
        compiler_params=pltpu.CompilerParams(
            collective_id=0, vmem_limit_bytes=41 * 1024 * 1024),
    )(xb, wb)
